# baseline (speedup 1.0000x reference)
import functools

import jax
import jax.numpy as jnp
from jax import lax
from jax.experimental import pallas as pl
from jax.experimental.pallas import tpu as pltpu
from jax.experimental.pallas import tpu_sc as plsc

LOSS_COEF = 0.01
BLK = 256
NW = 32


def _meta_body(x_ref, m_ref, wg_ref,
               d0_ref, d1_ref, g1_ref, g2_ref, be_ref, val_ref, loss_ref,
               csum_ref, *, E, G):
    S = x_ref.shape[0]
    C = 512
    x = x_ref[...]
    logits = lax.dot_general(x, wg_ref[...], (((1,), (1,)), ((), ())),
                             preferred_element_type=jnp.float32)
    idx8 = lax.broadcasted_iota(jnp.int32, (S, E), 1)
    m1 = jnp.max(logits, axis=1, keepdims=True)
    i1 = jnp.min(jnp.where(logits == m1, idx8, E), axis=1, keepdims=True)
    masked = jnp.where(idx8 == i1, -jnp.inf, logits)
    m2 = jnp.max(masked, axis=1, keepdims=True)
    i2 = jnp.min(jnp.where(masked == m2, idx8, E), axis=1, keepdims=True)
    b = jnp.exp(m2 - m1)
    den = 1.0 + b
    mk = m_ref[...].reshape(S, 1)
    g1v = mk / den
    g2v = mk * b / den
    g1_ref[...] = g1v
    g2_ref[...] = g2v
    oh1 = (idx8 == i1).astype(jnp.float32)
    oh2 = (idx8 == i2).astype(jnp.float32)

    imp = jnp.sum(oh1 * g1v + oh2 * g2v, axis=0, keepdims=True)
    mean = jnp.mean(imp, axis=1, keepdims=True)
    var = jnp.sum((imp - mean) ** 2, axis=1, keepdims=True) / (E - 1)
    loss_ref[...] = LOSS_COEF * var / (mean * mean + 1e-10)

    r = lax.broadcasted_iota(jnp.int32, (C, C), 0)
    c = lax.broadcasted_iota(jnp.int32, (C, C), 1)
    tri = (r >= c).astype(jnp.float32)
    nch = S // C
    chunks, tots = [], []
    for ci in range(2 * nch):
        src = oh1 if ci < nch else oh2
        off = (ci % nch) * C
        ohc = src[off:off + C, :]
        chunks.append(ohc)
        tots.append(jnp.sum(ohc, axis=0, keepdims=True))
    carry = jnp.zeros((1, E), jnp.float32)
    for ci in range(2 * nch):
        cs = jnp.dot(tri, chunks[ci], preferred_element_type=jnp.float32) + carry
        csum_ref[ci * C:(ci + 1) * C, :] = cs
        carry = carry + tots[ci]

    counts = carry
    padded = jnp.floor((counts + (BLK - 1)) * (1.0 / BLK)) * BLK
    ii = lax.broadcasted_iota(jnp.int32, (E, E), 0)
    jj = lax.broadcasted_iota(jnp.int32, (E, E), 1)
    triu = (ii < jj).astype(jnp.float32)
    offs = jnp.dot(padded, triu, preferred_element_type=jnp.float32)
    total = jnp.sum(padded, axis=1, keepdims=True)

    dest_all = csum_ref[...] - 1.0 + offs
    d0 = jnp.sum(oh1 * dest_all[0:S, :], axis=1, keepdims=True)
    d1 = jnp.sum(oh2 * dest_all[S:2 * S, :], axis=1, keepdims=True)
    d0_ref[...] = d0.astype(jnp.int32).reshape(S // 128, 128)
    d1_ref[...] = d1.astype(jnp.int32).reshape(S // 128, 128)

    gb = (lax.broadcasted_iota(jnp.int32, (G, E), 0) * BLK).astype(jnp.float32)
    gbc = jnp.minimum(gb, total - BLK)
    be = jnp.sum((gbc >= offs).astype(jnp.float32), axis=1, keepdims=True) - 1.0
    be_ref[...] = be.astype(jnp.int32)
    val_ref[...] = (gbc[:, 0:1] * (1.0 / BLK)).astype(jnp.int32)


def _routing_meta(xs, maskf, w_gate, E, G):
    S, D = xs.shape
    return pl.pallas_call(
        functools.partial(_meta_body, E=E, G=G),
        in_specs=[
            pl.BlockSpec((S, D), lambda: (0, 0)),
            pl.BlockSpec((1, S), lambda: (0, 0)),
            pl.BlockSpec((E, D), lambda: (0, 0)),
        ],
        out_specs=[
            pl.BlockSpec((S // 128, 128), lambda: (0, 0)),
            pl.BlockSpec((S // 128, 128), lambda: (0, 0)),
            pl.BlockSpec((S, 1), lambda: (0, 0)),
            pl.BlockSpec((S, 1), lambda: (0, 0)),
            pl.BlockSpec((G, 1), lambda: (0, 0)),
            pl.BlockSpec((G, 1), lambda: (0, 0)),
            pl.BlockSpec((1, 1), lambda: (0, 0)),
        ],
        out_shape=[
            jax.ShapeDtypeStruct((S // 128, 128), jnp.int32),
            jax.ShapeDtypeStruct((S // 128, 128), jnp.int32),
            jax.ShapeDtypeStruct((S, 1), jnp.float32),
            jax.ShapeDtypeStruct((S, 1), jnp.float32),
            jax.ShapeDtypeStruct((G, 1), jnp.int32),
            jax.ShapeDtypeStruct((G, 1), jnp.int32),
            jax.ShapeDtypeStruct((1, 1), jnp.float32),
        ],
        scratch_shapes=[pltpu.VMEM((2 * S, E), jnp.float32)],
    )(xs, maskf, w_gate)


def _make_dispatch(S, D, NPAD):
    CH = S // NW
    mesh = plsc.VectorSubcoreMesh(core_axis_name="c", subcore_axis_name="s")

    @functools.partial(
        pl.kernel, mesh=mesh,
        out_type=jax.ShapeDtypeStruct((NPAD, D), jnp.float32),
        scratch_types=[
            pltpu.VMEM((CH,), jnp.int32),
            pltpu.VMEM((CH, D), jnp.float32),
            pltpu.SemaphoreType.DMA,
        ],
    )
    def dispatch(x_hbm, d0_hbm, d1_hbm, xs_hbm, idx_v, rows_v, sem):
        wid = lax.axis_index("s") * 2 + lax.axis_index("c")
        base = wid * CH
        pltpu.sync_copy(x_hbm.at[pl.ds(base, CH)], rows_v)
        pltpu.sync_copy(d0_hbm.at[pl.ds(base, CH)], idx_v)
        pltpu.async_copy(rows_v, xs_hbm.at[idx_v], sem).wait()
        pltpu.sync_copy(d1_hbm.at[pl.ds(base, CH)], idx_v)
        pltpu.async_copy(rows_v, xs_hbm.at[idx_v], sem).wait()

    return dispatch


def _make_collect(S, D, NPAD):
    CH = S // NW
    mesh = plsc.VectorSubcoreMesh(core_axis_name="c", subcore_axis_name="s")

    @functools.partial(
        pl.kernel, mesh=mesh,
        out_type=(jax.ShapeDtypeStruct((S, D), jnp.float32),
                  jax.ShapeDtypeStruct((S, D), jnp.float32)),
        scratch_types=[
            pltpu.VMEM((CH,), jnp.int32),
            pltpu.VMEM((CH, D), jnp.float32),
            pltpu.SemaphoreType.DMA,
        ],
    )
    def collect(outp_hbm, d0_hbm, d1_hbm, r0_hbm, r1_hbm, idx_v, rows_v, sem):
        wid = lax.axis_index("s") * 2 + lax.axis_index("c")
        base = wid * CH
        pltpu.sync_copy(d0_hbm.at[pl.ds(base, CH)], idx_v)
        pltpu.async_copy(outp_hbm.at[idx_v], rows_v, sem).wait()
        pltpu.sync_copy(rows_v, r0_hbm.at[pl.ds(base, CH)])
        pltpu.sync_copy(d1_hbm.at[pl.ds(base, CH)], idx_v)
        pltpu.async_copy(outp_hbm.at[idx_v], rows_v, sem).wait()
        pltpu.sync_copy(rows_v, r1_hbm.at[pl.ds(base, CH)])

    return collect


def _gmm_body(be_ref, bidx_ref, xs_ref, W1_ref, b1_ref, W2_ref, b2_ref,
              out_ref):
    g = pl.program_id(0)

    @pl.when(bidx_ref[g, 0] == g)
    def _():
        h = jnp.dot(xs_ref[...], W1_ref[0],
                    preferred_element_type=jnp.float32) + b1_ref[0]
        h = jnp.maximum(h, 0.0)
        out_ref[...] = jnp.dot(h, W2_ref[0],
                               preferred_element_type=jnp.float32) + b2_ref[0]


def _grouped_mlp(xs_s, be, bidx, W1, b1r, W2, b2r, G):
    NPAD, D = xs_s.shape
    E, _, H = W1.shape
    return pl.pallas_call(
        _gmm_body,
        grid_spec=pltpu.PrefetchScalarGridSpec(
            num_scalar_prefetch=2,
            grid=(G,),
            in_specs=[
                pl.BlockSpec((BLK, D), lambda g, be, bidx: (bidx[g, 0], 0)),
                pl.BlockSpec((1, D, H), lambda g, be, bidx: (be[g, 0], 0, 0)),
                pl.BlockSpec((1, 1, H), lambda g, be, bidx: (be[g, 0], 0, 0)),
                pl.BlockSpec((1, H, D), lambda g, be, bidx: (be[g, 0], 0, 0)),
                pl.BlockSpec((1, 1, D), lambda g, be, bidx: (be[g, 0], 0, 0)),
            ],
            out_specs=pl.BlockSpec((BLK, D), lambda g, be, bidx: (bidx[g, 0], 0)),
        ),
        out_shape=jax.ShapeDtypeStruct((NPAD, D), jnp.float32),
        compiler_params=pltpu.CompilerParams(
            dimension_semantics=("arbitrary",),
        ),
    )(be, bidx, xs_s, W1, b1r, W2, b2r)


def _combine_body(x_ref, r0_ref, r1_ref, g1_ref, g2_ref, y_ref):
    z = g1_ref[...] * r0_ref[...] + g2_ref[...] * r1_ref[...]
    y_ref[...] = jax.nn.sigmoid(z) + x_ref[...]


def _combine(xs, r0, r1, g1, g2):
    S, D = xs.shape
    Sb = 512
    T = S // Sb
    specs = [
        pl.BlockSpec((Sb, D), lambda t: (t, 0)),
        pl.BlockSpec((Sb, D), lambda t: (t, 0)),
        pl.BlockSpec((Sb, D), lambda t: (t, 0)),
        pl.BlockSpec((Sb, 1), lambda t: (t, 0)),
        pl.BlockSpec((Sb, 1), lambda t: (t, 0)),
    ]
    return pl.pallas_call(
        _combine_body,
        grid=(T,),
        in_specs=specs,
        out_specs=pl.BlockSpec((Sb, D), lambda t: (t, 0)),
        out_shape=jax.ShapeDtypeStruct((S, D), jnp.float32),
    )(xs, r0, r1, g1, g2)


def kernel(x, mask, w_gate, W1, b1, W2, b2):
    B, S, D = x.shape
    E = w_gate.shape[1]
    H = W1.shape[2]
    G = 2 * S // BLK + E
    NPAD = G * BLK
    xs = x.reshape(S, D)
    maskf = mask.reshape(1, S).astype(jnp.float32)
    b1r = b1.reshape(E, 1, H)
    b2r = b2.reshape(E, 1, D)

    d0, d1, g1, g2, be, bidx, loss = _routing_meta(xs, maskf, w_gate.T, E, G)
    d0f = d0.reshape(S)
    d1f = d1.reshape(S)

    xs_sorted = _make_dispatch(S, D, NPAD)(xs, d0f, d1f)
    outp = _grouped_mlp(xs_sorted, be, bidx, W1, b1r, W2, b2r, G)
    r0, r1 = _make_collect(S, D, NPAD)(outp, d0f, d1f)
    y = _combine(xs, r0, r1, g1, g2)

    return y.reshape(B, S, D), loss[0, 0]


# speedup vs baseline: 1.0788x; 1.0788x over previous
import functools

import jax
import jax.numpy as jnp
from jax import lax
from jax.experimental import pallas as pl
from jax.experimental.pallas import tpu as pltpu
from jax.experimental.pallas import tpu_sc as plsc

LOSS_COEF = 0.01
BLK = 512
NW = 32


def _meta_body(x_ref, m_ref, wg_ref,
               d0_ref, d1_ref, g1_ref, g2_ref, be_ref, val_ref, loss_ref,
               csum_ref, *, E, G):
    S = x_ref.shape[0]
    C = 512
    x = x_ref[...]
    logits = lax.dot_general(x, wg_ref[...], (((1,), (1,)), ((), ())),
                             preferred_element_type=jnp.float32)
    idx8 = lax.broadcasted_iota(jnp.int32, (S, E), 1)
    m1 = jnp.max(logits, axis=1, keepdims=True)
    i1 = jnp.min(jnp.where(logits == m1, idx8, E), axis=1, keepdims=True)
    masked = jnp.where(idx8 == i1, -jnp.inf, logits)
    m2 = jnp.max(masked, axis=1, keepdims=True)
    i2 = jnp.min(jnp.where(masked == m2, idx8, E), axis=1, keepdims=True)
    b = jnp.exp(m2 - m1)
    den = 1.0 + b
    mk = m_ref[...].reshape(S, 1)
    g1v = mk / den
    g2v = mk * b / den
    g1_ref[...] = g1v
    g2_ref[...] = g2v
    oh1 = (idx8 == i1).astype(jnp.float32)
    oh2 = (idx8 == i2).astype(jnp.float32)

    imp = jnp.sum(oh1 * g1v + oh2 * g2v, axis=0, keepdims=True)
    mean = jnp.mean(imp, axis=1, keepdims=True)
    var = jnp.sum((imp - mean) ** 2, axis=1, keepdims=True) / (E - 1)
    loss_ref[...] = LOSS_COEF * var / (mean * mean + 1e-10)

    r = lax.broadcasted_iota(jnp.int32, (C, C), 0)
    c = lax.broadcasted_iota(jnp.int32, (C, C), 1)
    tri = (r >= c).astype(jnp.float32)
    nch = S // C
    chunks, tots = [], []
    for ci in range(2 * nch):
        src = oh1 if ci < nch else oh2
        off = (ci % nch) * C
        ohc = src[off:off + C, :]
        chunks.append(ohc)
        tots.append(jnp.sum(ohc, axis=0, keepdims=True))
    carry = jnp.zeros((1, E), jnp.float32)
    for ci in range(2 * nch):
        cs = jnp.dot(tri, chunks[ci], preferred_element_type=jnp.float32) + carry
        csum_ref[ci * C:(ci + 1) * C, :] = cs
        carry = carry + tots[ci]

    counts = carry
    padded = jnp.floor((counts + (BLK - 1)) * (1.0 / BLK)) * BLK
    ii = lax.broadcasted_iota(jnp.int32, (E, E), 0)
    jj = lax.broadcasted_iota(jnp.int32, (E, E), 1)
    triu = (ii < jj).astype(jnp.float32)
    offs = jnp.dot(padded, triu, preferred_element_type=jnp.float32)
    total = jnp.sum(padded, axis=1, keepdims=True)

    dest_all = csum_ref[...] - 1.0 + offs
    d0 = jnp.sum(oh1 * dest_all[0:S, :], axis=1, keepdims=True)
    d1 = jnp.sum(oh2 * dest_all[S:2 * S, :], axis=1, keepdims=True)
    d0_ref[...] = d0.astype(jnp.int32).reshape(S // 128, 128)
    d1_ref[...] = d1.astype(jnp.int32).reshape(S // 128, 128)

    gb = (lax.broadcasted_iota(jnp.int32, (G, E), 0) * BLK).astype(jnp.float32)
    gbc = jnp.minimum(gb, total - BLK)
    be = jnp.sum((gbc >= offs).astype(jnp.float32), axis=1, keepdims=True) - 1.0
    be_ref[...] = be.astype(jnp.int32)
    val_ref[...] = (gbc[:, 0:1] * (1.0 / BLK)).astype(jnp.int32)


def _routing_meta(xs, maskf, w_gate, E, G):
    S, D = xs.shape
    return pl.pallas_call(
        functools.partial(_meta_body, E=E, G=G),
        in_specs=[
            pl.BlockSpec((S, D), lambda: (0, 0)),
            pl.BlockSpec((1, S), lambda: (0, 0)),
            pl.BlockSpec((E, D), lambda: (0, 0)),
        ],
        out_specs=[
            pl.BlockSpec((S // 128, 128), lambda: (0, 0)),
            pl.BlockSpec((S // 128, 128), lambda: (0, 0)),
            pl.BlockSpec((S, 1), lambda: (0, 0)),
            pl.BlockSpec((S, 1), lambda: (0, 0)),
            pl.BlockSpec((G, 1), lambda: (0, 0)),
            pl.BlockSpec((G, 1), lambda: (0, 0)),
            pl.BlockSpec((1, 1), lambda: (0, 0)),
        ],
        out_shape=[
            jax.ShapeDtypeStruct((S // 128, 128), jnp.int32),
            jax.ShapeDtypeStruct((S // 128, 128), jnp.int32),
            jax.ShapeDtypeStruct((S, 1), jnp.float32),
            jax.ShapeDtypeStruct((S, 1), jnp.float32),
            jax.ShapeDtypeStruct((G, 1), jnp.int32),
            jax.ShapeDtypeStruct((G, 1), jnp.int32),
            jax.ShapeDtypeStruct((1, 1), jnp.float32),
        ],
        scratch_shapes=[pltpu.VMEM((2 * S, E), jnp.float32)],
    )(xs, maskf, w_gate)


def _make_dispatch(S, D, NPAD):
    CH = S // NW
    mesh = plsc.VectorSubcoreMesh(core_axis_name="c", subcore_axis_name="s")

    @functools.partial(
        pl.kernel, mesh=mesh,
        out_type=jax.ShapeDtypeStruct((NPAD, D), jnp.float32),
        scratch_types=[
            pltpu.VMEM((CH,), jnp.int32),
            pltpu.VMEM((CH,), jnp.int32),
            pltpu.VMEM((CH, D), jnp.float32),
            pltpu.SemaphoreType.DMA,
            pltpu.SemaphoreType.DMA,
        ],
    )
    def dispatch(x_hbm, d0_hbm, d1_hbm, xs_hbm, idx0_v, idx1_v, rows_v,
                 sem0, sem1):
        wid = lax.axis_index("s") * 2 + lax.axis_index("c")
        base = wid * CH
        pltpu.sync_copy(x_hbm.at[pl.ds(base, CH)], rows_v)
        pltpu.sync_copy(d0_hbm.at[pl.ds(base, CH)], idx0_v)
        pltpu.sync_copy(d1_hbm.at[pl.ds(base, CH)], idx1_v)
        cp0 = pltpu.async_copy(rows_v, xs_hbm.at[idx0_v], sem0)
        cp1 = pltpu.async_copy(rows_v, xs_hbm.at[idx1_v], sem1)
        cp0.wait()
        cp1.wait()

    return dispatch


def _make_collect(S, D, NPAD):
    CH = S // NW
    mesh = plsc.VectorSubcoreMesh(core_axis_name="c", subcore_axis_name="s")

    @functools.partial(
        pl.kernel, mesh=mesh,
        out_type=(jax.ShapeDtypeStruct((S, D), jnp.float32),
                  jax.ShapeDtypeStruct((S, D), jnp.float32)),
        scratch_types=[
            pltpu.VMEM((CH,), jnp.int32),
            pltpu.VMEM((CH, D), jnp.float32),
            pltpu.SemaphoreType.DMA,
        ],
    )
    def collect(outp_hbm, d0_hbm, d1_hbm, r0_hbm, r1_hbm, idx_v, rows_v, sem):
        wid = lax.axis_index("s") * 2 + lax.axis_index("c")
        base = wid * CH
        pltpu.sync_copy(d0_hbm.at[pl.ds(base, CH)], idx_v)
        pltpu.async_copy(outp_hbm.at[idx_v], rows_v, sem).wait()
        pltpu.sync_copy(rows_v, r0_hbm.at[pl.ds(base, CH)])
        pltpu.sync_copy(d1_hbm.at[pl.ds(base, CH)], idx_v)
        pltpu.async_copy(outp_hbm.at[idx_v], rows_v, sem).wait()
        pltpu.sync_copy(rows_v, r1_hbm.at[pl.ds(base, CH)])

    return collect


def _gmm_body(be_ref, bidx_ref, xs_ref, W1_ref, b1_ref, W2_ref, b2_ref,
              out_ref):
    g = pl.program_id(0)

    @pl.when(bidx_ref[g, 0] == g)
    def _():
        h = jnp.dot(xs_ref[...], W1_ref[0],
                    preferred_element_type=jnp.float32) + b1_ref[0]
        h = jnp.maximum(h, 0.0)
        out_ref[...] = jnp.dot(h, W2_ref[0],
                               preferred_element_type=jnp.float32) + b2_ref[0]


def _grouped_mlp(xs_s, be, bidx, W1, b1r, W2, b2r, G):
    NPAD, D = xs_s.shape
    E, _, H = W1.shape
    return pl.pallas_call(
        _gmm_body,
        grid_spec=pltpu.PrefetchScalarGridSpec(
            num_scalar_prefetch=2,
            grid=(G,),
            in_specs=[
                pl.BlockSpec((BLK, D), lambda g, be, bidx: (bidx[g, 0], 0)),
                pl.BlockSpec((1, D, H), lambda g, be, bidx: (be[g, 0], 0, 0)),
                pl.BlockSpec((1, 1, H), lambda g, be, bidx: (be[g, 0], 0, 0)),
                pl.BlockSpec((1, H, D), lambda g, be, bidx: (be[g, 0], 0, 0)),
                pl.BlockSpec((1, 1, D), lambda g, be, bidx: (be[g, 0], 0, 0)),
            ],
            out_specs=pl.BlockSpec((BLK, D), lambda g, be, bidx: (bidx[g, 0], 0)),
        ),
        out_shape=jax.ShapeDtypeStruct((NPAD, D), jnp.float32),
        compiler_params=pltpu.CompilerParams(
            dimension_semantics=("arbitrary",),
        ),
    )(be, bidx, xs_s, W1, b1r, W2, b2r)


def _combine_body(x_ref, r0_ref, r1_ref, g1_ref, g2_ref, y_ref):
    z = g1_ref[...] * r0_ref[...] + g2_ref[...] * r1_ref[...]
    y_ref[...] = jax.nn.sigmoid(z) + x_ref[...]


def _combine(xs, r0, r1, g1, g2):
    S, D = xs.shape
    Sb = 512
    T = S // Sb
    specs = [
        pl.BlockSpec((Sb, D), lambda t: (t, 0)),
        pl.BlockSpec((Sb, D), lambda t: (t, 0)),
        pl.BlockSpec((Sb, D), lambda t: (t, 0)),
        pl.BlockSpec((Sb, 1), lambda t: (t, 0)),
        pl.BlockSpec((Sb, 1), lambda t: (t, 0)),
    ]
    return pl.pallas_call(
        _combine_body,
        grid=(T,),
        in_specs=specs,
        out_specs=pl.BlockSpec((Sb, D), lambda t: (t, 0)),
        out_shape=jax.ShapeDtypeStruct((S, D), jnp.float32),
    )(xs, r0, r1, g1, g2)


def kernel(x, mask, w_gate, W1, b1, W2, b2):
    B, S, D = x.shape
    E = w_gate.shape[1]
    H = W1.shape[2]
    G = 2 * S // BLK + E
    NPAD = G * BLK
    xs = x.reshape(S, D)
    maskf = mask.reshape(1, S).astype(jnp.float32)
    b1r = b1.reshape(E, 1, H)
    b2r = b2.reshape(E, 1, D)

    d0, d1, g1, g2, be, bidx, loss = _routing_meta(xs, maskf, w_gate.T, E, G)
    d0f = d0.reshape(S)
    d1f = d1.reshape(S)

    xs_sorted = _make_dispatch(S, D, NPAD)(xs, d0f, d1f)
    outp = _grouped_mlp(xs_sorted, be, bidx, W1, b1r, W2, b2r, G)
    r0, r1 = _make_collect(S, D, NPAD)(outp, d0f, d1f)
    y = _combine(xs, r0, r1, g1, g2)

    return y.reshape(B, S, D), loss[0, 0]
